# Initial kernel scaffold; baseline (speedup 1.0000x reference)
import jax
import jax.numpy as jnp
from jax.experimental import pallas as pl


def kernel(x, W1, b1, W2, b2, expert_weights):
    raise NotImplementedError("write your pallas kernel here")


# speedup vs baseline: 1.9186x; 1.9186x over previous
import functools

import jax
import jax.numpy as jnp
from jax.experimental import pallas as pl

_NUM_TOKENS = 32768
_INPUT_DIM = 768
_HIDDEN = 256
_NUM_EXPERTS = 64
_TILE = 1024


def _router_kernel(x_ref, w1_ref, b1_ref, w2_ref, b2_ref, wout_ref, iout_ref):
    x = x_ref[...]
    h = jax.lax.dot_general(
        x, w1_ref[...], (((1,), (0,)), ((), ())),
        preferred_element_type=jnp.float32,
    )
    h = jnp.maximum(h + b1_ref[...], 0.0)
    logits = jax.lax.dot_general(
        h, w2_ref[...], (((1,), (0,)), ((), ())),
        preferred_element_type=jnp.float32,
    )
    logits = logits + b2_ref[...]

    iota = jax.lax.broadcasted_iota(jnp.int32, logits.shape, 1)
    m1 = jnp.max(logits, axis=-1, keepdims=True)
    i1 = jnp.min(jnp.where(logits == m1, iota, _NUM_EXPERTS),
                 axis=-1, keepdims=True)
    masked = jnp.where(iota == i1, -jnp.inf, logits)
    m2 = jnp.max(masked, axis=-1, keepdims=True)
    i2 = jnp.min(jnp.where(masked == m2, iota, _NUM_EXPERTS),
                 axis=-1, keepdims=True)

    denom = jnp.sum(jnp.exp(logits - m1), axis=-1, keepdims=True)
    w_top1 = 1.0 / denom
    w_top2 = jnp.exp(m2 - m1) / denom

    wout_ref[...] = jnp.concatenate([w_top1, w_top2], axis=-1)
    iout_ref[...] = jnp.concatenate([i1, i2], axis=-1)


@functools.partial(jax.jit, static_argnames=())
def kernel(x, W1, b1, W2, b2, expert_weights):
    n_tokens = x.shape[0]
    bias2 = (b2 + expert_weights).reshape(1, _NUM_EXPERTS)
    b1r = b1.reshape(1, _HIDDEN)
    grid = (n_tokens // _TILE,)
    weights, indices = pl.pallas_call(
        _router_kernel,
        grid=grid,
        in_specs=[
            pl.BlockSpec((_TILE, _INPUT_DIM), lambda i: (i, 0)),
            pl.BlockSpec((_INPUT_DIM, _HIDDEN), lambda i: (0, 0)),
            pl.BlockSpec((1, _HIDDEN), lambda i: (0, 0)),
            pl.BlockSpec((_HIDDEN, _NUM_EXPERTS), lambda i: (0, 0)),
            pl.BlockSpec((1, _NUM_EXPERTS), lambda i: (0, 0)),
        ],
        out_specs=[
            pl.BlockSpec((_TILE, 2), lambda i: (i, 0)),
            pl.BlockSpec((_TILE, 2), lambda i: (i, 0)),
        ],
        out_shape=[
            jax.ShapeDtypeStruct((n_tokens, 2), jnp.float32),
            jax.ShapeDtypeStruct((n_tokens, 2), jnp.int32),
        ],
    )(x, W1, b1r, W2, bias2)
    return weights, indices


# speedup vs baseline: 2.0185x; 1.0521x over previous
import functools

import jax
import jax.numpy as jnp
from jax.experimental import pallas as pl

_NUM_TOKENS = 32768
_INPUT_DIM = 768
_HIDDEN = 256
_NUM_EXPERTS = 64
_TILE = 1024


def _router_kernel(x_ref, w1_ref, b1_ref, w2_ref, b2_ref, wout_ref, iout_ref):
    x = x_ref[...]
    h = jax.lax.dot_general(
        x, w1_ref[...], (((1,), (0,)), ((), ())),
        preferred_element_type=jnp.float32,
    )
    h = jnp.maximum(h + b1_ref[...], 0.0)
    logits = jax.lax.dot_general(
        h, w2_ref[...], (((1,), (0,)), ((), ())),
        preferred_element_type=jnp.float32,
    )
    logits = logits + b2_ref[...]

    iota = jax.lax.broadcasted_iota(jnp.int32, logits.shape, 1).astype(
        jnp.float32)
    big = jnp.float32(_NUM_EXPERTS)
    m1 = jnp.max(logits, axis=-1, keepdims=True)
    i1 = jnp.min(jnp.where(logits == m1, iota, big), axis=-1, keepdims=True)
    masked = jnp.where(iota == i1, -jnp.inf, logits)
    m2 = jnp.max(masked, axis=-1, keepdims=True)
    i2 = jnp.min(jnp.where(masked == m2, iota, big), axis=-1, keepdims=True)

    denom = jnp.sum(jnp.exp(logits - m1), axis=-1, keepdims=True)
    recip = 1.0 / denom
    w_top1 = recip
    w_top2 = jnp.exp(m2 - m1) * recip

    wout_ref[...] = jnp.concatenate([w_top1, w_top2], axis=-1)
    iout_ref[...] = jnp.concatenate([i1, i2], axis=-1).astype(jnp.int32)


@functools.partial(jax.jit, static_argnames=())
def kernel(x, W1, b1, W2, b2, expert_weights):
    n_tokens = x.shape[0]
    bias2 = (b2 + expert_weights).reshape(1, _NUM_EXPERTS)
    b1r = b1.reshape(1, _HIDDEN)
    grid = (n_tokens // _TILE,)
    weights, indices = pl.pallas_call(
        _router_kernel,
        grid=grid,
        in_specs=[
            pl.BlockSpec((_TILE, _INPUT_DIM), lambda i: (i, 0)),
            pl.BlockSpec((_INPUT_DIM, _HIDDEN), lambda i: (0, 0)),
            pl.BlockSpec((1, _HIDDEN), lambda i: (0, 0)),
            pl.BlockSpec((_HIDDEN, _NUM_EXPERTS), lambda i: (0, 0)),
            pl.BlockSpec((1, _NUM_EXPERTS), lambda i: (0, 0)),
        ],
        out_specs=[
            pl.BlockSpec((_TILE, 2), lambda i: (i, 0)),
            pl.BlockSpec((_TILE, 2), lambda i: (i, 0)),
        ],
        out_shape=[
            jax.ShapeDtypeStruct((n_tokens, 2), jnp.float32),
            jax.ShapeDtypeStruct((n_tokens, 2), jnp.int32),
        ],
    )(x, W1, b1r, W2, bias2)
    return weights, indices


# speedup vs baseline: 2.3692x; 1.1737x over previous
import functools

import jax
import jax.numpy as jnp
from jax.experimental import pallas as pl

_NUM_TOKENS = 32768
_INPUT_DIM = 768
_HIDDEN = 256
_NUM_EXPERTS = 64
_TILE = 2048


def _router_kernel(x_ref, w1_ref, b1_ref, w2_ref, b2_ref, wout_ref, iout_ref):
    x = x_ref[...]
    h = jax.lax.dot_general(
        x, w1_ref[...], (((1,), (0,)), ((), ())),
        preferred_element_type=jnp.float32,
    )
    h = jnp.maximum(h + b1_ref[...], 0.0)
    logits = jax.lax.dot_general(
        h, w2_ref[...], (((1,), (0,)), ((), ())),
        preferred_element_type=jnp.float32,
    )
    logits = logits + b2_ref[...]

    iota = jax.lax.broadcasted_iota(jnp.int32, logits.shape, 1).astype(
        jnp.float32)
    big = jnp.float32(_NUM_EXPERTS)
    m1 = jnp.max(logits, axis=-1, keepdims=True)
    i1 = jnp.min(jnp.where(logits == m1, iota, big), axis=-1, keepdims=True)
    masked = jnp.where(iota == i1, -jnp.inf, logits)
    m2 = jnp.max(masked, axis=-1, keepdims=True)
    i2 = jnp.min(jnp.where(masked == m2, iota, big), axis=-1, keepdims=True)

    denom = jnp.sum(jnp.exp(logits - m1), axis=-1, keepdims=True)
    recip = 1.0 / denom
    w_top1 = recip
    w_top2 = jnp.exp(m2 - m1) * recip

    wout_ref[...] = jnp.concatenate([w_top1, w_top2], axis=-1)
    iout_ref[...] = jnp.concatenate([i1, i2], axis=-1).astype(jnp.int32)


@functools.partial(jax.jit, static_argnames=())
def kernel(x, W1, b1, W2, b2, expert_weights):
    n_tokens = x.shape[0]
    bias2 = (b2 + expert_weights).reshape(1, _NUM_EXPERTS)
    b1r = b1.reshape(1, _HIDDEN)
    grid = (n_tokens // _TILE,)
    weights, indices = pl.pallas_call(
        _router_kernel,
        grid=grid,
        in_specs=[
            pl.BlockSpec((_TILE, _INPUT_DIM), lambda i: (i, 0)),
            pl.BlockSpec((_INPUT_DIM, _HIDDEN), lambda i: (0, 0)),
            pl.BlockSpec((1, _HIDDEN), lambda i: (0, 0)),
            pl.BlockSpec((_HIDDEN, _NUM_EXPERTS), lambda i: (0, 0)),
            pl.BlockSpec((1, _NUM_EXPERTS), lambda i: (0, 0)),
        ],
        out_specs=[
            pl.BlockSpec((_TILE, 2), lambda i: (i, 0)),
            pl.BlockSpec((_TILE, 2), lambda i: (i, 0)),
        ],
        out_shape=[
            jax.ShapeDtypeStruct((n_tokens, 2), jnp.float32),
            jax.ShapeDtypeStruct((n_tokens, 2), jnp.int32),
        ],
    )(x, W1, b1r, W2, bias2)
    return weights, indices


# speedup vs baseline: 2.5181x; 1.0628x over previous
import functools

import jax
import jax.numpy as jnp
from jax.experimental import pallas as pl

_NUM_TOKENS = 32768
_INPUT_DIM = 768
_HIDDEN = 256
_NUM_EXPERTS = 64
_TILE = 4096


def _router_kernel(x_ref, w1_ref, b1_ref, w2_ref, b2_ref, wout_ref, iout_ref):
    x = x_ref[...]
    h = jax.lax.dot_general(
        x, w1_ref[...], (((1,), (0,)), ((), ())),
        preferred_element_type=jnp.float32,
    )
    h = jnp.maximum(h + b1_ref[...], 0.0)
    logits = jax.lax.dot_general(
        h, w2_ref[...], (((1,), (0,)), ((), ())),
        preferred_element_type=jnp.float32,
    )
    logits = logits + b2_ref[...]

    iota = jax.lax.broadcasted_iota(jnp.int32, logits.shape, 1).astype(
        jnp.float32)
    big = jnp.float32(_NUM_EXPERTS)
    m1 = jnp.max(logits, axis=-1, keepdims=True)
    i1 = jnp.min(jnp.where(logits == m1, iota, big), axis=-1, keepdims=True)
    masked = jnp.where(iota == i1, -jnp.inf, logits)
    m2 = jnp.max(masked, axis=-1, keepdims=True)
    i2 = jnp.min(jnp.where(masked == m2, iota, big), axis=-1, keepdims=True)

    denom = jnp.sum(jnp.exp(logits - m1), axis=-1, keepdims=True)
    recip = 1.0 / denom
    w_top1 = recip
    w_top2 = jnp.exp(m2 - m1) * recip

    wout_ref[...] = jnp.concatenate([w_top1, w_top2], axis=-1)
    iout_ref[...] = jnp.concatenate([i1, i2], axis=-1).astype(jnp.int32)


@functools.partial(jax.jit, static_argnames=())
def kernel(x, W1, b1, W2, b2, expert_weights):
    n_tokens = x.shape[0]
    bias2 = (b2 + expert_weights).reshape(1, _NUM_EXPERTS)
    b1r = b1.reshape(1, _HIDDEN)
    grid = (n_tokens // _TILE,)
    weights, indices = pl.pallas_call(
        _router_kernel,
        grid=grid,
        in_specs=[
            pl.BlockSpec((_TILE, _INPUT_DIM), lambda i: (i, 0)),
            pl.BlockSpec((_INPUT_DIM, _HIDDEN), lambda i: (0, 0)),
            pl.BlockSpec((1, _HIDDEN), lambda i: (0, 0)),
            pl.BlockSpec((_HIDDEN, _NUM_EXPERTS), lambda i: (0, 0)),
            pl.BlockSpec((1, _NUM_EXPERTS), lambda i: (0, 0)),
        ],
        out_specs=[
            pl.BlockSpec((_TILE, 2), lambda i: (i, 0)),
            pl.BlockSpec((_TILE, 2), lambda i: (i, 0)),
        ],
        out_shape=[
            jax.ShapeDtypeStruct((n_tokens, 2), jnp.float32),
            jax.ShapeDtypeStruct((n_tokens, 2), jnp.int32),
        ],
    )(x, W1, b1r, W2, bias2)
    return weights, indices
